# baseline (speedup 1.0000x reference)
import functools

import jax
import jax.numpy as jnp
from jax import lax
from jax.experimental import pallas as pl
from jax.experimental.pallas import tpu as pltpu
from jax.experimental.pallas import tpu_sc as plsc

S_BLK = 512
GCH = 128
P_BLK = 512
NCB = 32


def _sc_scatter_rows(table, didx3, *, seqlen, n_hashes):
    info = plsc.get_sparse_core_info()
    ncores, nsub = info.num_cores, info.num_subcores
    nw = ncores * nsub
    ntasks, kch, gch = didx3.shape
    tpw = ntasks // nw
    assert tpw * nw == ntasks
    chunk_rows = kch * gch
    n_rows, r = table.shape
    m_out = n_rows * n_hashes
    mesh = plsc.VectorSubcoreMesh(core_axis_name="c", subcore_axis_name="s")

    @functools.partial(
        pl.kernel, mesh=mesh,
        out_type=jax.ShapeDtypeStruct((m_out, r), jnp.float32),
        scratch_types=[pltpu.VMEM((kch, gch), jnp.int32),
                       pltpu.VMEM((2, gch, r), jnp.float32),
                       pltpu.SemaphoreType.DMA,
                       pltpu.SemaphoreType.DMA],
    )
    def k(t_hbm, idx_hbm, o_hbm, idx_v, buf, rsem, wsem):
        wid = lax.axis_index("s") * ncores + lax.axis_index("c")
        for ti in range(tpw):
            task = wid * tpw + ti
            flat0 = task * chunk_rows
            head = flat0 // (n_hashes * seqlen)
            t0 = flat0 % seqlen
            src0 = head * seqlen + t0
            pltpu.sync_copy(idx_hbm.at[task], idx_v)
            pltpu.async_copy(t_hbm.at[pl.ds(src0, gch)], buf.at[0], rsem)

            def body(j, carry):
                del carry
                cur = lax.rem(j, 2)
                nxt = lax.rem(j + 1, 2)

                @pl.when(j + 1 < kch)
                def _():
                    pltpu.async_copy(t_hbm.at[pl.ds(src0 + (j + 1) * gch, gch)],
                                     buf.at[nxt], rsem)

                pltpu.make_async_copy(t_hbm.at[pl.ds(src0, gch)], buf.at[0],
                                      rsem).wait()
                pltpu.async_copy(buf.at[cur], o_hbm.at[idx_v.at[j]], wsem).wait()
                return 0

            lax.fori_loop(0, kch, body, 0)

    return k(table, didx3)


def _sc_gather_rows(table, gidx3):
    info = plsc.get_sparse_core_info()
    ncores, nsub = info.num_cores, info.num_subcores
    nw, kch, gch = gidx3.shape
    assert nw == ncores * nsub
    rows = kch * gch
    m = nw * rows
    r = table.shape[1]
    mesh = plsc.VectorSubcoreMesh(core_axis_name="c", subcore_axis_name="s")

    @functools.partial(
        pl.kernel, mesh=mesh,
        out_type=jax.ShapeDtypeStruct((m, r), jnp.float32),
        scratch_types=[pltpu.VMEM((kch, gch), jnp.int32),
                       pltpu.VMEM((2, gch, r), jnp.float32),
                       pltpu.SemaphoreType.DMA,
                       pltpu.SemaphoreType.DMA],
    )
    def k(t_hbm, idx_hbm, o_hbm, idx_v, buf, gsem, wsem):
        wid = lax.axis_index("s") * ncores + lax.axis_index("c")
        pltpu.sync_copy(idx_hbm.at[wid], idx_v)
        base = wid * rows

        pltpu.async_copy(t_hbm.at[idx_v.at[0]], buf.at[0], gsem)

        def body(j, carry):
            del carry
            cur = lax.rem(j, 2)
            nxt = lax.rem(j + 1, 2)

            @pl.when(j + 1 < kch)
            def _():
                pltpu.async_copy(t_hbm.at[idx_v.at[j + 1]], buf.at[nxt], gsem)

            pltpu.make_async_copy(t_hbm.at[idx_v.at[0]], buf.at[0], gsem).wait()
            pltpu.async_copy(buf.at[cur],
                             o_hbm.at[pl.ds(base + j * gch, gch)], wsem).wait()
            return 0

        lax.fori_loop(0, kch, body, 0)

    return k(table, gidx3)


def _proj_hash_body(xq_ref, xkv_ref, wq_ref, bq_ref, wv_ref, bv_ref, rot_ref,
                    qv_ref, bkt_ref, hist_ref, hist_scr,
                    *, n_buckets, hd, n_sb):
    xq = xq_ref[...]
    xkv = xkv_ref[...]
    wq = wq_ref[0]
    wv = wv_ref[0]
    q = jnp.dot(xq, wq, preferred_element_type=jnp.float32) + bq_ref[0]
    v = jnp.dot(xkv, wv, preferred_element_type=jnp.float32) + bv_ref[0]
    qv_ref[0, :, 0:hd] = q
    qv_ref[0, :, hd:2 * hd] = v

    rot = rot_ref[...]
    r = jnp.dot(q, rot, preferred_element_type=jnp.float32)
    n_rot = n_buckets // 2
    nb2 = 2 * n_buckets
    sb = pl.program_id(0)
    hh = pl.program_id(1)
    cnt = jnp.zeros((1, nb2), jnp.int32)
    for j in range(2):
        rj = r[:, j * n_rot:(j + 1) * n_rot]
        m = jnp.max(jnp.maximum(rj, -rj), axis=1, keepdims=True)
        iota = jax.lax.broadcasted_iota(jnp.int32, (S_BLK, n_rot), 1)
        a1 = jnp.min(jnp.where(rj == m, iota, n_buckets), axis=1, keepdims=True)
        a2 = jnp.min(jnp.where(-rj == m, iota + n_rot, n_buckets), axis=1,
                     keepdims=True)
        bkt = jnp.minimum(a1, a2) + j * n_buckets
        bkt_ref[0, j] = bkt
        lanes = jax.lax.broadcasted_iota(jnp.int32, (S_BLK, nb2), 1)
        cnt = cnt + jnp.sum((bkt == lanes).astype(jnp.int32), axis=0,
                            keepdims=True)
    old = hist_scr[pl.ds(hh, 1), :]
    new = jnp.where(sb == 0, cnt, old + cnt)
    hist_scr[pl.ds(hh, 1), :] = new
    hist_ref[0] = new


def _proj_hash(xq, xkv, wq_t, bq3, wv_t, bv3, rot, *, n_buckets):
    seqlen, d = xq.shape
    h, _, hd = wq_t.shape
    n_sb = seqlen // S_BLK
    grid = (n_sb, h)
    body = functools.partial(_proj_hash_body, n_buckets=n_buckets, hd=hd,
                             n_sb=n_sb)
    return pl.pallas_call(
        body,
        grid=grid,
        in_specs=[
            pl.BlockSpec((S_BLK, d), lambda sb, hh: (sb, 0)),
            pl.BlockSpec((S_BLK, d), lambda sb, hh: (sb, 0)),
            pl.BlockSpec((1, d, hd), lambda sb, hh: (hh, 0, 0)),
            pl.BlockSpec((1, 1, hd), lambda sb, hh: (hh, 0, 0)),
            pl.BlockSpec((1, d, hd), lambda sb, hh: (hh, 0, 0)),
            pl.BlockSpec((1, 1, hd), lambda sb, hh: (hh, 0, 0)),
            pl.BlockSpec((hd, n_buckets), lambda sb, hh: (0, 0)),
        ],
        out_specs=[
            pl.BlockSpec((1, S_BLK, 2 * hd), lambda sb, hh: (hh, sb, 0)),
            pl.BlockSpec((1, 2, S_BLK, 1), lambda sb, hh: (hh, 0, sb, 0)),
            pl.BlockSpec((1, 1, 2 * n_buckets), lambda sb, hh: (hh, 0, 0)),
        ],
        out_shape=[
            jax.ShapeDtypeStruct((h, seqlen, 2 * hd), jnp.float32),
            jax.ShapeDtypeStruct((h, 2, seqlen, 1), jnp.int32),
            jax.ShapeDtypeStruct((h, 1, 2 * n_buckets), jnp.int32),
        ],
        scratch_shapes=[pltpu.VMEM((h, 2 * n_buckets), jnp.int32)],
    )(xq, xkv, wq_t, bq3, wv_t, bv3, rot)


def _pos_body(bkt_ref, hist_ref, ltri_ref, pos_ref, *, nb2, nblk):
    h0 = hist_ref[0]
    incl = h0
    k = 1
    while k < nb2:
        incl = incl + jnp.concatenate(
            [jnp.zeros((1, k), jnp.int32), incl[:, :nb2 - k]], axis=1)
        k *= 2
    start = (incl - h0).astype(jnp.float32)
    ltri = ltri_ref[...]

    def blk_step(i, base):
        b = bkt_ref[0, i]
        lanes = jax.lax.broadcasted_iota(jnp.int32, (P_BLK, nb2), 1)
        onehot = (b == lanes).astype(jnp.float32)
        pre = jnp.dot(ltri, onehot, preferred_element_type=jnp.float32)
        rank_incl = jnp.sum(pre * onehot, axis=1, keepdims=True)
        basev = jnp.sum(base * onehot, axis=1, keepdims=True)
        pos_ref[0, i] = (basev + rank_incl).astype(jnp.int32) - 1
        return base + jnp.sum(onehot, axis=0, keepdims=True)

    lax.fori_loop(0, nblk, blk_step, start)


def _sort_positions(bkt, hist, *, n_buckets):
    h, n_hashes, seqlen, _ = bkt.shape
    n_total = n_hashes * seqlen
    nblk = n_total // P_BLK
    nb2 = 2 * n_buckets
    bkt4 = bkt.reshape(h, nblk, P_BLK, 1)
    ri = jnp.arange(P_BLK, dtype=jnp.int32)
    ltri = (ri[:, None] >= ri[None, :]).astype(jnp.float32)
    body = functools.partial(_pos_body, nb2=nb2, nblk=nblk)
    pos4 = pl.pallas_call(
        body,
        grid=(h,),
        in_specs=[
            pl.BlockSpec((1, nblk, P_BLK, 1), lambda hh: (hh, 0, 0, 0)),
            pl.BlockSpec((1, 1, nb2), lambda hh: (hh, 0, 0)),
            pl.BlockSpec((P_BLK, P_BLK), lambda hh: (0, 0)),
        ],
        out_specs=pl.BlockSpec((1, nblk, P_BLK, 1), lambda hh: (hh, 0, 0, 0)),
        out_shape=jax.ShapeDtypeStruct((h, nblk, P_BLK, 1), jnp.int32),
    )(bkt4, hist, ltri)
    return pos4.reshape(h, n_total)


def _attn_body(qvc_ref, qvp_ref, x_ref, *, scale, hd):
    def one_chunk(qvc, qvp):
        q = qvc[:, 0:hd]
        kv_cat = jnp.concatenate([qvc, qvp], axis=0)
        kcat = kv_cat[:, 0:hd]
        vcat = kv_cat[:, hd:2 * hd]
        dn = (((1,), (1,)), ((), ()))
        dd = jax.lax.dot_general(q, kcat, dn, preferred_element_type=jnp.float32)
        dd = dd * scale
        m = jnp.max(dd, axis=1, keepdims=True)
        e = jnp.exp(dd - m)
        s = jnp.sum(e, axis=1, keepdims=True)
        lse = m + jnp.log(s)
        x = jnp.dot(e * (1.0 / s), vcat, preferred_element_type=jnp.float32)
        return x, lse

    for cc in range(NCB):
        qvc = qvc_ref[0, cc]
        qvp = qvc_ref[0, cc - 1] if cc > 0 else qvp_ref[0, NCB - 1]
        x, lse = one_chunk(qvc, qvp)
        x_ref[0, cc, :, 0:hd] = x
        x_ref[0, cc, :, hd:2 * hd] = jnp.broadcast_to(lse, x.shape)


def _chunk_attn(sqv, *, scale):
    h, nc, ch, hd2 = sqv.shape
    hd = hd2 // 2
    ng = nc // NCB
    body = functools.partial(_attn_body, scale=scale, hd=hd)
    return pl.pallas_call(
        body,
        grid=(h, ng),
        in_specs=[
            pl.BlockSpec((1, NCB, ch, hd2), lambda hh, g: (hh, g, 0, 0)),
            pl.BlockSpec((1, NCB, ch, hd2), lambda hh, g: (hh, (g + ng - 1) % ng, 0, 0)),
        ],
        out_specs=pl.BlockSpec((1, NCB, ch, hd2), lambda hh, g: (hh, g, 0, 0)),
        out_shape=jax.ShapeDtypeStruct((h, nc, ch, hd2), jnp.float32),
    )(sqv, sqv)


def _combine_body(o_ref, wout_ref, bout_ref, out_ref, *, h, hd):
    ws = []
    for hh in range(h):
        a0 = o_ref[hh, 0, :, hd:hd + 1]
        a1 = o_ref[hh, 1, :, hd:hd + 1]
        p0 = 1.0 / (1.0 + jnp.exp(a1 - a0))
        p1 = 1.0 / (1.0 + jnp.exp(a0 - a1))
        ws.append(o_ref[hh, 0, :, 0:hd] * p0 + o_ref[hh, 1, :, 0:hd] * p1)
    w = jnp.concatenate(ws, axis=1)
    acc = jnp.dot(w, wout_ref[...], preferred_element_type=jnp.float32)
    out_ref[...] = acc + bout_ref[...]


def _combine_proj(o, wout, bout2):
    h, _, seqlen, hd2 = o.shape
    hd = hd2 // 2
    d = wout.shape[-1]
    wout2 = wout.reshape(h * hd, d)
    n_sb = seqlen // S_BLK
    body = functools.partial(_combine_body, h=h, hd=hd)
    return pl.pallas_call(
        body,
        grid=(n_sb,),
        in_specs=[
            pl.BlockSpec((h, 2, S_BLK, hd2), lambda sb: (0, 0, sb, 0)),
            pl.BlockSpec((h * hd, d), lambda sb: (0, 0)),
            pl.BlockSpec((1, d), lambda sb: (0, 0)),
        ],
        out_specs=pl.BlockSpec((S_BLK, d), lambda sb: (sb, 0)),
        out_shape=jax.ShapeDtypeStruct((seqlen, d), jnp.float32),
    )(o, wout2, bout2)


def _single_batch(xq, xkv, Wq, bq, Wv, bv, Wout, bout, *, n_buckets, n_hashes):
    seqlen, d = xq.shape
    h, hd = bq.shape
    n_total = n_hashes * seqlen

    rot = jax.random.normal(jax.random.PRNGKey(0), (hd, n_hashes, n_buckets // 2)
                            ).astype(jnp.float32).reshape(hd, n_hashes * (n_buckets // 2))

    qv, bkt, hist = _proj_hash(
        xq, xkv,
        Wq.transpose(1, 0, 2), bq.reshape(h, 1, hd),
        Wv.transpose(1, 0, 2), bv.reshape(h, 1, hd),
        rot, n_buckets=n_buckets)

    pos = _sort_positions(bkt, hist, n_buckets=n_buckets)

    info = plsc.get_sparse_core_info()
    n_workers = info.num_cores * info.num_subcores
    head_off = jnp.arange(h, dtype=jnp.int32)[:, None]
    gidx = head_off * n_total + pos

    ntasks = 3 * n_workers
    didx3 = gidx.reshape(ntasks, -1, GCH)
    sqv = _sc_scatter_rows(qv.reshape(h * seqlen, 2 * hd), didx3,
                           seqlen=seqlen, n_hashes=n_hashes)

    nc = n_hashes * n_buckets
    ch = n_total // nc
    x_aug = _chunk_attn(sqv.reshape(h, nc, ch, 2 * hd), scale=float(hd) ** 0.5)

    gidx_bwd = gidx.reshape(n_workers, -1, GCH)
    o = _sc_gather_rows(x_aug.reshape(h * n_total, 2 * hd), gidx_bwd)

    out2d = _combine_proj(o.reshape(h, n_hashes, seqlen, 2 * hd),
                          Wout, bout.reshape(1, d))
    return out2d


def kernel(inputs_q, inputs_kv, Wq, bq, Wv, bv, Wout, bout):
    b = inputs_q.shape[0]
    outs = [
        _single_batch(inputs_q[i], inputs_kv[i], Wq, bq, Wv, bv, Wout, bout,
                      n_buckets=64, n_hashes=2)
        for i in range(b)
    ]
    return jnp.stack(outs, axis=0)


# speedup vs baseline: 1.0371x; 1.0371x over previous
import functools

import jax
import jax.numpy as jnp
from jax import lax
from jax.experimental import pallas as pl
from jax.experimental.pallas import tpu as pltpu
from jax.experimental.pallas import tpu_sc as plsc

S_BLK = 512
GCH = 128
P_BLK = 512
SUB_BLK = 128
NCB = 16


def _sc_scatter_rows(table, didx3, *, seqlen, n_hashes):
    info = plsc.get_sparse_core_info()
    ncores, nsub = info.num_cores, info.num_subcores
    nw = ncores * nsub
    ntasks, kch, gch = didx3.shape
    tpw = ntasks // nw
    assert tpw * nw == ntasks
    chunk_rows = kch * gch
    n_rows, r = table.shape
    m_out = n_rows * n_hashes
    mesh = plsc.VectorSubcoreMesh(core_axis_name="c", subcore_axis_name="s")

    @functools.partial(
        pl.kernel, mesh=mesh,
        out_type=jax.ShapeDtypeStruct((m_out, r), jnp.float32),
        scratch_types=[pltpu.VMEM((kch, gch), jnp.int32),
                       pltpu.VMEM((2, gch, r), jnp.float32),
                       pltpu.SemaphoreType.DMA,
                       pltpu.SemaphoreType.DMA],
    )
    def k(t_hbm, idx_hbm, o_hbm, idx_v, buf, rsem, wsem):
        wid = lax.axis_index("s") * ncores + lax.axis_index("c")
        for ti in range(tpw):
            task = wid * tpw + ti
            flat0 = task * chunk_rows
            head = flat0 // (n_hashes * seqlen)
            t0 = flat0 % seqlen
            src0 = head * seqlen + t0
            pltpu.sync_copy(idx_hbm.at[task], idx_v)
            pltpu.async_copy(t_hbm.at[pl.ds(src0, gch)], buf.at[0], rsem)

            def body(j, carry):
                del carry
                cur = lax.rem(j, 2)
                nxt = lax.rem(j + 1, 2)

                @pl.when(j + 1 < kch)
                def _():
                    pltpu.async_copy(t_hbm.at[pl.ds(src0 + (j + 1) * gch, gch)],
                                     buf.at[nxt], rsem)

                pltpu.make_async_copy(t_hbm.at[pl.ds(src0, gch)], buf.at[0],
                                      rsem).wait()
                pltpu.async_copy(buf.at[cur], o_hbm.at[idx_v.at[j]], wsem).wait()
                return 0

            lax.fori_loop(0, kch, body, 0)

    return k(table, didx3)


def _sc_gather_rows(table, gidx3):
    info = plsc.get_sparse_core_info()
    ncores, nsub = info.num_cores, info.num_subcores
    nw, kch, gch = gidx3.shape
    assert nw == ncores * nsub
    rows = kch * gch
    m = nw * rows
    r = table.shape[1]
    mesh = plsc.VectorSubcoreMesh(core_axis_name="c", subcore_axis_name="s")

    @functools.partial(
        pl.kernel, mesh=mesh,
        out_type=jax.ShapeDtypeStruct((m, r), jnp.float32),
        scratch_types=[pltpu.VMEM((kch, gch), jnp.int32),
                       pltpu.VMEM((2, gch, r), jnp.float32),
                       pltpu.SemaphoreType.DMA,
                       pltpu.SemaphoreType.DMA],
    )
    def k(t_hbm, idx_hbm, o_hbm, idx_v, buf, gsem, wsem):
        wid = lax.axis_index("s") * ncores + lax.axis_index("c")
        pltpu.sync_copy(idx_hbm.at[wid], idx_v)
        base = wid * rows

        pltpu.async_copy(t_hbm.at[idx_v.at[0]], buf.at[0], gsem)

        def body(j, carry):
            del carry
            cur = lax.rem(j, 2)
            nxt = lax.rem(j + 1, 2)

            @pl.when(j + 1 < kch)
            def _():
                pltpu.async_copy(t_hbm.at[idx_v.at[j + 1]], buf.at[nxt], gsem)

            pltpu.make_async_copy(t_hbm.at[idx_v.at[0]], buf.at[0], gsem).wait()
            pltpu.async_copy(buf.at[cur],
                             o_hbm.at[pl.ds(base + j * gch, gch)], wsem).wait()
            return 0

        lax.fori_loop(0, kch, body, 0)

    return k(table, gidx3)


def _proj_hash_body(xq_ref, xkv_ref, wq_ref, bq_ref, wv_ref, bv_ref, rot_ref,
                    qv_ref, bkt_ref, hist_ref, hist_scr,
                    *, n_buckets, hd, n_sb):
    xq = xq_ref[...]
    xkv = xkv_ref[...]
    wq = wq_ref[0]
    wv = wv_ref[0]
    q = jnp.dot(xq, wq, preferred_element_type=jnp.float32) + bq_ref[0]
    v = jnp.dot(xkv, wv, preferred_element_type=jnp.float32) + bv_ref[0]
    qv_ref[0, :, 0:hd] = q
    qv_ref[0, :, hd:2 * hd] = v

    rot = rot_ref[...]
    r = jnp.dot(q, rot, preferred_element_type=jnp.float32)
    n_rot = n_buckets // 2
    nb2 = 2 * n_buckets
    sb = pl.program_id(0)
    hh = pl.program_id(1)
    cnt = jnp.zeros((1, nb2), jnp.int32)
    for j in range(2):
        rj = r[:, j * n_rot:(j + 1) * n_rot]
        m = jnp.max(jnp.maximum(rj, -rj), axis=1, keepdims=True)
        iota = jax.lax.broadcasted_iota(jnp.int32, (S_BLK, n_rot), 1)
        a1 = jnp.min(jnp.where(rj == m, iota, n_buckets), axis=1, keepdims=True)
        a2 = jnp.min(jnp.where(-rj == m, iota + n_rot, n_buckets), axis=1,
                     keepdims=True)
        bkt = jnp.minimum(a1, a2) + j * n_buckets
        bkt_ref[0, j] = bkt
        lanes = jax.lax.broadcasted_iota(jnp.int32, (S_BLK, nb2), 1)
        cnt = cnt + jnp.sum((bkt == lanes).astype(jnp.int32), axis=0,
                            keepdims=True)
    old = hist_scr[pl.ds(hh, 1), :]
    new = jnp.where(sb == 0, cnt, old + cnt)
    hist_scr[pl.ds(hh, 1), :] = new
    hist_ref[0] = new


def _proj_hash(xq, xkv, wq_t, bq3, wv_t, bv3, rot, *, n_buckets):
    seqlen, d = xq.shape
    h, _, hd = wq_t.shape
    n_sb = seqlen // S_BLK
    grid = (n_sb, h)
    body = functools.partial(_proj_hash_body, n_buckets=n_buckets, hd=hd,
                             n_sb=n_sb)
    return pl.pallas_call(
        body,
        grid=grid,
        in_specs=[
            pl.BlockSpec((S_BLK, d), lambda sb, hh: (sb, 0)),
            pl.BlockSpec((S_BLK, d), lambda sb, hh: (sb, 0)),
            pl.BlockSpec((1, d, hd), lambda sb, hh: (hh, 0, 0)),
            pl.BlockSpec((1, 1, hd), lambda sb, hh: (hh, 0, 0)),
            pl.BlockSpec((1, d, hd), lambda sb, hh: (hh, 0, 0)),
            pl.BlockSpec((1, 1, hd), lambda sb, hh: (hh, 0, 0)),
            pl.BlockSpec((hd, n_buckets), lambda sb, hh: (0, 0)),
        ],
        out_specs=[
            pl.BlockSpec((1, S_BLK, 2 * hd), lambda sb, hh: (hh, sb, 0)),
            pl.BlockSpec((1, 2, S_BLK, 1), lambda sb, hh: (hh, 0, sb, 0)),
            pl.BlockSpec((1, 1, 2 * n_buckets), lambda sb, hh: (hh, 0, 0)),
        ],
        out_shape=[
            jax.ShapeDtypeStruct((h, seqlen, 2 * hd), jnp.float32),
            jax.ShapeDtypeStruct((h, 2, seqlen, 1), jnp.int32),
            jax.ShapeDtypeStruct((h, 1, 2 * n_buckets), jnp.int32),
        ],
        scratch_shapes=[pltpu.VMEM((h, 2 * n_buckets), jnp.int32)],
    )(xq, xkv, wq_t, bq3, wv_t, bv3, rot)


def _pos_body(bkt_ref, hist_ref, ltri_ref, pos_ref, *, nb2, nblk):
    h0 = hist_ref[0]
    incl = h0
    k = 1
    while k < nb2:
        incl = incl + jnp.concatenate(
            [jnp.zeros((1, k), jnp.int32), incl[:, :nb2 - k]], axis=1)
        k *= 2
    start = (incl - h0).astype(jnp.float32)
    ltri = ltri_ref[...].astype(jnp.bfloat16)

    nsub = P_BLK // SUB_BLK

    def blk_step(i, base):
        bb = bkt_ref[0, i]
        for s in range(nsub):
            b = bb[s * SUB_BLK:(s + 1) * SUB_BLK]
            lanes = jax.lax.broadcasted_iota(jnp.int32, (SUB_BLK, nb2), 1)
            onehot = (b == lanes).astype(jnp.float32)
            pre = jnp.dot(ltri, onehot.astype(jnp.bfloat16),
                          preferred_element_type=jnp.float32)
            rank_incl = jnp.sum(pre * onehot, axis=1, keepdims=True)
            basev = jnp.sum(base * onehot, axis=1, keepdims=True)
            pos_ref[0, i, s * SUB_BLK:(s + 1) * SUB_BLK] = (
                (basev + rank_incl).astype(jnp.int32) - 1)
            base = base + jnp.sum(onehot, axis=0, keepdims=True)
        return base

    lax.fori_loop(0, nblk, blk_step, start)


def _sort_positions(bkt, hist, *, n_buckets):
    h, n_hashes, seqlen, _ = bkt.shape
    n_total = n_hashes * seqlen
    nblk = n_total // P_BLK
    nb2 = 2 * n_buckets
    bkt4 = bkt.reshape(h, nblk, P_BLK, 1)
    ri = jnp.arange(SUB_BLK, dtype=jnp.int32)
    ltri = (ri[:, None] >= ri[None, :]).astype(jnp.float32)
    body = functools.partial(_pos_body, nb2=nb2, nblk=nblk)
    pos4 = pl.pallas_call(
        body,
        grid=(h,),
        in_specs=[
            pl.BlockSpec((1, nblk, P_BLK, 1), lambda hh: (hh, 0, 0, 0)),
            pl.BlockSpec((1, 1, nb2), lambda hh: (hh, 0, 0)),
            pl.BlockSpec((SUB_BLK, SUB_BLK), lambda hh: (0, 0)),
        ],
        out_specs=pl.BlockSpec((1, nblk, P_BLK, 1), lambda hh: (hh, 0, 0, 0)),
        out_shape=jax.ShapeDtypeStruct((h, nblk, P_BLK, 1), jnp.int32),
    )(bkt4, hist, ltri)
    return pos4.reshape(h, n_total)


def _attn_body(qvc_ref, qvp_ref, x_ref, *, scale, hd):
    def one_chunk(qvc, qvp):
        q = qvc[:, 0:hd]
        kv_cat = jnp.concatenate([qvc, qvp], axis=0)
        kcat = kv_cat[:, 0:hd]
        vcat = kv_cat[:, hd:2 * hd]
        dn = (((1,), (1,)), ((), ()))
        dd = jax.lax.dot_general(q, kcat, dn, preferred_element_type=jnp.float32)
        dd = dd * scale
        m = jnp.max(dd, axis=1, keepdims=True)
        e = jnp.exp(dd - m)
        s = jnp.sum(e, axis=1, keepdims=True)
        lse = m + jnp.log(s)
        x = jnp.dot(e * (1.0 / s), vcat, preferred_element_type=jnp.float32)
        return x, lse

    for cc in range(NCB):
        qvc = qvc_ref[0, cc]
        qvp = qvc_ref[0, cc - 1] if cc > 0 else qvp_ref[0, NCB - 1]
        x, lse = one_chunk(qvc, qvp)
        x_ref[0, cc, :, 0:hd] = x
        x_ref[0, cc, :, hd:2 * hd] = jnp.broadcast_to(lse, x.shape)


def _chunk_attn(sqv, *, scale):
    h, nc, ch, hd2 = sqv.shape
    hd = hd2 // 2
    ng = nc // NCB
    body = functools.partial(_attn_body, scale=scale, hd=hd)
    return pl.pallas_call(
        body,
        grid=(h, ng),
        in_specs=[
            pl.BlockSpec((1, NCB, ch, hd2), lambda hh, g: (hh, g, 0, 0)),
            pl.BlockSpec((1, NCB, ch, hd2), lambda hh, g: (hh, (g + ng - 1) % ng, 0, 0)),
        ],
        out_specs=pl.BlockSpec((1, NCB, ch, hd2), lambda hh, g: (hh, g, 0, 0)),
        out_shape=jax.ShapeDtypeStruct((h, nc, ch, hd2), jnp.float32),
    )(sqv, sqv)


def _combine_body(o_ref, wout_ref, bout_ref, out_ref, *, h, hd):
    ws = []
    for hh in range(h):
        a0 = o_ref[hh, 0, :, hd:hd + 1]
        a1 = o_ref[hh, 1, :, hd:hd + 1]
        p0 = 1.0 / (1.0 + jnp.exp(a1 - a0))
        p1 = 1.0 / (1.0 + jnp.exp(a0 - a1))
        ws.append(o_ref[hh, 0, :, 0:hd] * p0 + o_ref[hh, 1, :, 0:hd] * p1)
    w = jnp.concatenate(ws, axis=1)
    acc = jnp.dot(w, wout_ref[...], preferred_element_type=jnp.float32)
    out_ref[...] = acc + bout_ref[...]


def _combine_proj(o, wout, bout2):
    h, _, seqlen, hd2 = o.shape
    hd = hd2 // 2
    d = wout.shape[-1]
    wout2 = wout.reshape(h * hd, d)
    n_sb = seqlen // S_BLK
    body = functools.partial(_combine_body, h=h, hd=hd)
    return pl.pallas_call(
        body,
        grid=(n_sb,),
        in_specs=[
            pl.BlockSpec((h, 2, S_BLK, hd2), lambda sb: (0, 0, sb, 0)),
            pl.BlockSpec((h * hd, d), lambda sb: (0, 0)),
            pl.BlockSpec((1, d), lambda sb: (0, 0)),
        ],
        out_specs=pl.BlockSpec((S_BLK, d), lambda sb: (sb, 0)),
        out_shape=jax.ShapeDtypeStruct((seqlen, d), jnp.float32),
    )(o, wout2, bout2)


def _single_batch(xq, xkv, Wq, bq, Wv, bv, Wout, bout, *, n_buckets, n_hashes):
    seqlen, d = xq.shape
    h, hd = bq.shape
    n_total = n_hashes * seqlen

    rot = jax.random.normal(jax.random.PRNGKey(0), (hd, n_hashes, n_buckets // 2)
                            ).astype(jnp.float32).reshape(hd, n_hashes * (n_buckets // 2))

    qv, bkt, hist = _proj_hash(
        xq, xkv,
        Wq.transpose(1, 0, 2), bq.reshape(h, 1, hd),
        Wv.transpose(1, 0, 2), bv.reshape(h, 1, hd),
        rot, n_buckets=n_buckets)

    pos = _sort_positions(bkt, hist, n_buckets=n_buckets)

    info = plsc.get_sparse_core_info()
    n_workers = info.num_cores * info.num_subcores
    head_off = jnp.arange(h, dtype=jnp.int32)[:, None]
    gidx = head_off * n_total + pos

    ntasks = 3 * n_workers
    didx3 = gidx.reshape(ntasks, -1, GCH)
    sqv = _sc_scatter_rows(qv.reshape(h * seqlen, 2 * hd), didx3,
                           seqlen=seqlen, n_hashes=n_hashes)

    nc = n_hashes * n_buckets
    ch = n_total // nc
    x_aug = _chunk_attn(sqv.reshape(h, nc, ch, 2 * hd), scale=float(hd) ** 0.5)

    gidx_bwd = gidx.reshape(n_workers, -1, GCH)
    o = _sc_gather_rows(x_aug.reshape(h * n_total, 2 * hd), gidx_bwd)

    out2d = _combine_proj(o.reshape(h, n_hashes, seqlen, 2 * hd),
                          Wout, bout.reshape(1, d))
    return out2d


def kernel(inputs_q, inputs_kv, Wq, bq, Wv, bv, Wout, bout):
    b = inputs_q.shape[0]
    outs = [
        _single_batch(inputs_q[i], inputs_kv[i], Wq, bq, Wv, bv, Wout, bout,
                      n_buckets=64, n_hashes=2)
        for i in range(b)
    ]
    return jnp.stack(outs, axis=0)
